# Initial kernel scaffold; baseline (speedup 1.0000x reference)
import jax
import jax.numpy as jnp
from jax.experimental import pallas as pl


def kernel(x, edge_index, y, W1_rel, b1_rel, W1_root, W2_rel, b2_rel, W2_root):
    raise NotImplementedError("write your pallas kernel here")


# speedup vs baseline: 8.5843x; 8.5843x over previous
import functools

import jax
import jax.numpy as jnp
from jax import lax
from jax.experimental import pallas as pl
from jax.experimental.pallas import tpu as pltpu
from jax.experimental.pallas import tpu_sc as plsc

N = 10000
E = 320000
D = 128
H = 32
C = 10
CP = 16
NPAD = 10240
NC = 2
NS = 16
NW = NC * NS
EB = 128
NB_PER_TILE = 8 * -(-E // (NW * EB * 8))
EPAD = NW * EB * NB_PER_TILE
ROWS_PER_TILE = NPAD // NS


def _tc_project(x, wa, wb):
    def body(x_ref, wa_ref, wb_ref, oa_ref, ob_ref):
        xv = x_ref[...]
        dn = (((1,), (1,)), ((), ()))
        oa_ref[...] = lax.dot_general(xv, wa_ref[...], dn,
                                      preferred_element_type=jnp.float32)
        ob_ref[...] = lax.dot_general(xv, wb_ref[...], dn,
                                      preferred_element_type=jnp.float32)
    m = x.shape[0]
    return pl.pallas_call(
        body,
        out_shape=(jax.ShapeDtypeStruct((m, wa.shape[0]), jnp.float32),
                   jax.ShapeDtypeStruct((m, wb.shape[0]), jnp.float32)),
    )(x, wa, wb)


def _tc_layer2(p0, p1, xroot, b1, w2rel, w2root):
    def body(p0_ref, p1_ref, xroot_ref, b1_ref, wr_ref, wo_ref,
             hr_ref, hroot_ref):
        h = jnp.maximum(
            p0_ref[...] + p1_ref[...] + xroot_ref[...] + b1_ref[...], 0.0)
        dn = (((1,), (1,)), ((), ()))
        hr_ref[...] = lax.dot_general(h, wr_ref[...], dn,
                                      preferred_element_type=jnp.float32)
        hroot_ref[...] = lax.dot_general(h, wo_ref[...], dn,
                                         preferred_element_type=jnp.float32)
    m = p0.shape[0]
    return pl.pallas_call(
        body,
        out_shape=(jax.ShapeDtypeStruct((m, CP), jnp.float32),
                   jax.ShapeDtypeStruct((m, CP), jnp.float32)),
    )(p0, p1, xroot, b1, w2rel, w2root)


def _tc_loss(p0, p1, hroot, b2, y2d):
    def body(p0_ref, p1_ref, hroot_ref, b2_ref, y_ref, o_ref):
        logits = p0_ref[...] + p1_ref[...] + hroot_ref[...] + b2_ref[...]
        col = lax.broadcasted_iota(jnp.int32, logits.shape, 1)
        lm = jnp.where(col < C, logits, -1e30)
        mx = jnp.max(lm, axis=1, keepdims=True)
        ex = jnp.exp(lm - mx)
        lse = jnp.log(jnp.sum(ex, axis=1, keepdims=True)) + mx
        picked = jnp.sum(jnp.where(col == y_ref[...], lm, 0.0),
                         axis=1, keepdims=True)
        nll = lse - picked
        row = lax.broadcasted_iota(jnp.int32, nll.shape, 0)
        nll = jnp.where(row < N, nll, 0.0)
        o_ref[...] = (jnp.sum(nll) * (1.0 / N)).reshape(1, 1)
    return pl.pallas_call(
        body,
        out_shape=jax.ShapeDtypeStruct((1, 1), jnp.float32),
    )(p0, p1, hroot, b2, y2d)


def _make_sc_segsum(width):
    mesh = plsc.VectorSubcoreMesh(
        core_axis_name="c", subcore_axis_name="s",
        num_cores=NC, num_subcores=NS)

    @functools.partial(
        pl.kernel, mesh=mesh,
        out_type=jax.ShapeDtypeStruct((NC, NPAD, width), jnp.float32),
        scratch_types=[
            pltpu.VMEM((NB_PER_TILE, EB), jnp.int32),
            pltpu.VMEM((NB_PER_TILE, EB), jnp.int32),
            pltpu.VMEM((EB, width), jnp.float32),
            pltpu.VMEM((ROWS_PER_TILE, width), jnp.float32),
            pltpu.VMEM_SHARED((NPAD, width), jnp.float32),
            pltpu.SemaphoreType.DMA,
        ],
        compiler_params=pltpu.CompilerParams(use_tc_tiling_on_sc=False),
    )
    def k(table_hbm, src_hbm, dst_hbm, zeros_hbm, out_hbm,
          src_v, dst_v, rows_v, stage_v, agg_sh, sem):
        cid = lax.axis_index("c")
        sid = lax.axis_index("s")
        wid = cid * NS + sid
        r0 = sid * ROWS_PER_TILE
        pltpu.sync_copy(zeros_hbm, stage_v)
        pltpu.sync_copy(stage_v, agg_sh.at[pl.ds(r0, ROWS_PER_TILE)])
        b0 = wid * NB_PER_TILE
        pltpu.sync_copy(src_hbm.at[pl.ds(b0, NB_PER_TILE)], src_v)
        pltpu.sync_copy(dst_hbm.at[pl.ds(b0, NB_PER_TILE)], dst_v)
        plsc.subcore_barrier()

        def body(j, carry):
            pltpu.async_copy(table_hbm.at[src_v.at[j]], rows_v, sem).wait()
            pltpu.sync_copy(rows_v, agg_sh.at[dst_v.at[j]], add=True)
            return carry
        lax.fori_loop(0, NB_PER_TILE, body, 0)

        plsc.subcore_barrier()
        pltpu.sync_copy(agg_sh.at[pl.ds(r0, ROWS_PER_TILE)], stage_v)
        pltpu.sync_copy(stage_v, out_hbm.at[cid, pl.ds(r0, ROWS_PER_TILE)])

    return k


_make_sc_segsum = functools.lru_cache(maxsize=None)(_make_sc_segsum)


def kernel(x, edge_index, y, W1_rel, b1_rel, W1_root, W2_rel, b2_rel, W2_root):
    x_p = jnp.pad(x, ((0, NPAD - N), (0, 0)))
    pad_e = EPAD - E
    src_p = jnp.concatenate(
        [edge_index[0], jnp.full((pad_e,), N, jnp.int32)]).reshape(-1, EB)
    dst_p = jnp.concatenate(
        [edge_index[1], jnp.full((pad_e,), N, jnp.int32)]).reshape(-1, EB)
    y2d = jnp.pad(y, (0, NPAD - N)).reshape(NPAD, 1)
    b1_2d = b1_rel.reshape(1, H)
    w2rel_p = jnp.pad(W2_rel, ((0, CP - C), (0, 0)))
    w2root_p = jnp.pad(W2_root, ((0, CP - C), (0, 0)))
    b2_2d = jnp.pad(b2_rel, (0, CP - C)).reshape(1, CP)
    zeros_h = jnp.zeros((ROWS_PER_TILE, H), jnp.float32)
    zeros_c = jnp.zeros((ROWS_PER_TILE, CP), jnp.float32)

    xr, xroot = _tc_project(x_p, W1_rel, W1_root)
    part1 = _make_sc_segsum(H)(xr, src_p, dst_p, zeros_h)
    hr, hroot = _tc_layer2(part1[0], part1[1], xroot, b1_2d,
                           w2rel_p, w2root_p)
    part2 = _make_sc_segsum(CP)(hr, src_p, dst_p, zeros_c)
    loss2d = _tc_loss(part2[0], part2[1], hroot, b2_2d, y2d)
    return (loss2d[0, 0],)


# speedup vs baseline: 10.5398x; 1.2278x over previous
import functools

import jax
import jax.numpy as jnp
from jax import lax
from jax.experimental import pallas as pl
from jax.experimental.pallas import tpu as pltpu
from jax.experimental.pallas import tpu_sc as plsc

N = 10000
E = 320000
D = 128
H = 32
C = 10
CP = 16
NPAD = 10240
NC = 2
NS = 16
NW = NC * NS
EB = 128
NBUF = 4
NB_PER_TILE = 8 * -(-E // (NW * EB * 8))
EPAD = NW * EB * NB_PER_TILE
ROWS_PER_TILE = NPAD // NS


def _tc_project(x, wa, wb):
    def body(x_ref, wa_ref, wb_ref, oa_ref, ob_ref):
        xv = x_ref[...]
        dn = (((1,), (1,)), ((), ()))
        oa_ref[...] = lax.dot_general(xv, wa_ref[...], dn,
                                      preferred_element_type=jnp.float32)
        ob_ref[...] = lax.dot_general(xv, wb_ref[...], dn,
                                      preferred_element_type=jnp.float32)
    m = x.shape[0]
    return pl.pallas_call(
        body,
        out_shape=(jax.ShapeDtypeStruct((m, wa.shape[0]), jnp.float32),
                   jax.ShapeDtypeStruct((m, wb.shape[0]), jnp.float32)),
    )(x, wa, wb)


def _tc_layer2(p0, p1, xroot, b1, w2rel, w2root):
    def body(p0_ref, p1_ref, xroot_ref, b1_ref, wr_ref, wo_ref,
             hr_ref, hroot_ref):
        h = jnp.maximum(
            p0_ref[...] + p1_ref[...] + xroot_ref[...] + b1_ref[...], 0.0)
        dn = (((1,), (1,)), ((), ()))
        hr_ref[...] = lax.dot_general(h, wr_ref[...], dn,
                                      preferred_element_type=jnp.float32)
        hroot_ref[...] = lax.dot_general(h, wo_ref[...], dn,
                                         preferred_element_type=jnp.float32)
    m = p0.shape[0]
    return pl.pallas_call(
        body,
        out_shape=(jax.ShapeDtypeStruct((m, CP), jnp.float32),
                   jax.ShapeDtypeStruct((m, CP), jnp.float32)),
    )(p0, p1, xroot, b1, w2rel, w2root)


def _tc_loss(p0, p1, hroot, b2, y2d):
    def body(p0_ref, p1_ref, hroot_ref, b2_ref, y_ref, o_ref):
        logits = p0_ref[...] + p1_ref[...] + hroot_ref[...] + b2_ref[...]
        col = lax.broadcasted_iota(jnp.int32, logits.shape, 1)
        lm = jnp.where(col < C, logits, -1e30)
        mx = jnp.max(lm, axis=1, keepdims=True)
        ex = jnp.exp(lm - mx)
        lse = jnp.log(jnp.sum(ex, axis=1, keepdims=True)) + mx
        picked = jnp.sum(jnp.where(col == y_ref[...], lm, 0.0),
                         axis=1, keepdims=True)
        nll = lse - picked
        row = lax.broadcasted_iota(jnp.int32, nll.shape, 0)
        nll = jnp.where(row < N, nll, 0.0)
        o_ref[...] = (jnp.sum(nll) * (1.0 / N)).reshape(1, 1)
    return pl.pallas_call(
        body,
        out_shape=jax.ShapeDtypeStruct((1, 1), jnp.float32),
    )(p0, p1, hroot, b2, y2d)


def _make_sc_segsum(width):
    mesh = plsc.VectorSubcoreMesh(
        core_axis_name="c", subcore_axis_name="s",
        num_cores=NC, num_subcores=NS)

    @functools.partial(
        pl.kernel, mesh=mesh,
        out_type=jax.ShapeDtypeStruct((NC, NPAD, width), jnp.float32),
        scratch_types=[
            pltpu.VMEM((NB_PER_TILE, EB), jnp.int32),
            pltpu.VMEM((NB_PER_TILE, EB), jnp.int32),
            [pltpu.VMEM((EB, width), jnp.float32)] * NBUF,
            pltpu.VMEM((ROWS_PER_TILE, width), jnp.float32),
            pltpu.VMEM_SHARED((NPAD, width), jnp.float32),
            [pltpu.SemaphoreType.DMA] * NBUF,
            [pltpu.SemaphoreType.DMA] * NBUF,
        ],
        compiler_params=pltpu.CompilerParams(use_tc_tiling_on_sc=False),
    )
    def k(table_hbm, src_hbm, dst_hbm, zeros_hbm, out_hbm,
          src_v, dst_v, rows, stage_v, agg_sh, gsem, ssem):
        cid = lax.axis_index("c")
        sid = lax.axis_index("s")
        wid = cid * NS + sid
        r0 = sid * ROWS_PER_TILE
        pltpu.sync_copy(zeros_hbm, stage_v)
        pltpu.sync_copy(stage_v, agg_sh.at[pl.ds(r0, ROWS_PER_TILE)])
        b0 = wid * NB_PER_TILE
        pltpu.sync_copy(src_hbm.at[pl.ds(b0, NB_PER_TILE)], src_v)
        pltpu.sync_copy(dst_hbm.at[pl.ds(b0, NB_PER_TILE)], dst_v)
        plsc.subcore_barrier()

        def gather(j, b):
            pltpu.async_copy(table_hbm.at[src_v.at[j]], rows[b], gsem[b])

        def wait_gather(j, b):
            pltpu.make_async_copy(
                table_hbm.at[src_v.at[j]], rows[b], gsem[b]).wait()

        def scatter(j, b):
            pltpu.async_copy(
                rows[b], agg_sh.at[dst_v.at[j]], ssem[b], add=True)

        def wait_scatter(j, b):
            pltpu.make_async_copy(
                rows[b], agg_sh.at[dst_v.at[j]], ssem[b]).wait()

        for b in range(NBUF):
            gather(b, b)
        @pl.loop(0, NB_PER_TILE - NBUF, step=NBUF)
        def _(i):
            for b in range(NBUF):
                wait_gather(i + b, b)
                scatter(i + b, b)
            for b in range(NBUF):
                wait_scatter(i + b, b)
                gather(i + NBUF + b, b)
        for b in range(NBUF):
            j = NB_PER_TILE - NBUF + b
            wait_gather(j, b)
            scatter(j, b)
        for b in range(NBUF):
            wait_scatter(NB_PER_TILE - NBUF + b, b)

        plsc.subcore_barrier()
        pltpu.sync_copy(agg_sh.at[pl.ds(r0, ROWS_PER_TILE)], stage_v)
        pltpu.sync_copy(stage_v, out_hbm.at[cid, pl.ds(r0, ROWS_PER_TILE)])

    return k


_make_sc_segsum = functools.lru_cache(maxsize=None)(_make_sc_segsum)


def kernel(x, edge_index, y, W1_rel, b1_rel, W1_root, W2_rel, b2_rel, W2_root):
    x_p = jnp.pad(x, ((0, NPAD - N), (0, 0)))
    pad_e = EPAD - E
    src_p = jnp.concatenate(
        [edge_index[0], jnp.full((pad_e,), N, jnp.int32)]).reshape(-1, EB)
    dst_p = jnp.concatenate(
        [edge_index[1], jnp.full((pad_e,), N, jnp.int32)]).reshape(-1, EB)
    y2d = jnp.pad(y, (0, NPAD - N)).reshape(NPAD, 1)
    b1_2d = b1_rel.reshape(1, H)
    w2rel_p = jnp.pad(W2_rel, ((0, CP - C), (0, 0)))
    w2root_p = jnp.pad(W2_root, ((0, CP - C), (0, 0)))
    b2_2d = jnp.pad(b2_rel, (0, CP - C)).reshape(1, CP)
    zeros_h = jnp.zeros((ROWS_PER_TILE, H), jnp.float32)
    zeros_c = jnp.zeros((ROWS_PER_TILE, CP), jnp.float32)

    xr, xroot = _tc_project(x_p, W1_rel, W1_root)
    part1 = _make_sc_segsum(H)(xr, src_p, dst_p, zeros_h)
    hr, hroot = _tc_layer2(part1[0], part1[1], xroot, b1_2d,
                           w2rel_p, w2root_p)
    part2 = _make_sc_segsum(CP)(hr, src_p, dst_p, zeros_c)
    loss2d = _tc_loss(part2[0], part2[1], hroot, b2_2d, y2d)
    return (loss2d[0, 0],)


# speedup vs baseline: 19.0951x; 1.8117x over previous
import functools

import jax
import jax.numpy as jnp
from jax import lax
from jax.experimental import pallas as pl
from jax.experimental.pallas import tpu as pltpu
from jax.experimental.pallas import tpu_sc as plsc

N = 10000
E = 320000
D = 128
H = 32
C = 10
CP = 16
NPAD = 10240
NC = 2
NS = 16
NW = NC * NS
EB = 128
NBUF = 4
NB_PER_TILE = 8 * -(-E // (NW * EB * 8))
EPAD = NW * EB * NB_PER_TILE
ROWS_PER_TILE = NPAD // NS


def _tc_project(x, wa, wb):
    def body(x_ref, wa_ref, wb_ref, oa_ref, ob_ref):
        xv = x_ref[...]
        dn = (((1,), (1,)), ((), ()))
        oa_ref[...] = lax.dot_general(xv, wa_ref[...], dn,
                                      preferred_element_type=jnp.float32)
        ob_ref[...] = lax.dot_general(xv, wb_ref[...], dn,
                                      preferred_element_type=jnp.float32)
    m = x.shape[0]
    return pl.pallas_call(
        body,
        out_shape=(jax.ShapeDtypeStruct((m, wa.shape[0]), jnp.float32),
                   jax.ShapeDtypeStruct((m, wb.shape[0]), jnp.float32)),
    )(x, wa, wb)


def _tc_layer2(p0, p1, xroot, b1, w2rel, w2root):
    def body(p0_ref, p1_ref, xroot_ref, b1_ref, wr_ref, wo_ref,
             hr_ref, hroot_ref):
        h = jnp.maximum(
            p0_ref[...] + p1_ref[...] + xroot_ref[...] + b1_ref[...], 0.0)
        dn = (((1,), (1,)), ((), ()))
        hr_ref[...] = lax.dot_general(h, wr_ref[...], dn,
                                      preferred_element_type=jnp.float32)
        hroot_ref[...] = lax.dot_general(h, wo_ref[...], dn,
                                         preferred_element_type=jnp.float32)
    m = p0.shape[0]
    return pl.pallas_call(
        body,
        out_shape=(jax.ShapeDtypeStruct((m, CP), jnp.float32),
                   jax.ShapeDtypeStruct((m, CP), jnp.float32)),
    )(p0, p1, xroot, b1, w2rel, w2root)


def _tc_loss(p0, p1, hroot, b2, y2d):
    def body(p0_ref, p1_ref, hroot_ref, b2_ref, y_ref, o_ref):
        logits = p0_ref[...] + p1_ref[...] + hroot_ref[...] + b2_ref[...]
        col = lax.broadcasted_iota(jnp.int32, logits.shape, 1)
        lm = jnp.where(col < C, logits, -1e30)
        mx = jnp.max(lm, axis=1, keepdims=True)
        ex = jnp.exp(lm - mx)
        lse = jnp.log(jnp.sum(ex, axis=1, keepdims=True)) + mx
        picked = jnp.sum(jnp.where(col == y_ref[...], lm, 0.0),
                         axis=1, keepdims=True)
        nll = lse - picked
        row = lax.broadcasted_iota(jnp.int32, nll.shape, 0)
        nll = jnp.where(row < N, nll, 0.0)
        o_ref[...] = (jnp.sum(nll) * (1.0 / N)).reshape(1, 1)
    return pl.pallas_call(
        body,
        out_shape=jax.ShapeDtypeStruct((1, 1), jnp.float32),
    )(p0, p1, hroot, b2, y2d)


def _make_sc_segsum(width):
    mesh = plsc.VectorSubcoreMesh(
        core_axis_name="c", subcore_axis_name="s",
        num_cores=NC, num_subcores=NS)

    @functools.partial(
        pl.kernel, mesh=mesh,
        out_type=jax.ShapeDtypeStruct((NC, NPAD, width), jnp.float32),
        scratch_types=[
            pltpu.VMEM((NB_PER_TILE, EB), jnp.int32),
            pltpu.VMEM((NB_PER_TILE, EB), jnp.int32),
            [pltpu.VMEM((EB, width), jnp.float32)] * NBUF,
            pltpu.VMEM((ROWS_PER_TILE, width), jnp.float32),
            pltpu.VMEM_SHARED((NPAD, width), jnp.float32),
            [pltpu.SemaphoreType.DMA] * NBUF,
            [pltpu.SemaphoreType.DMA] * NBUF,
        ],
        compiler_params=pltpu.CompilerParams(use_tc_tiling_on_sc=False),
    )
    def k(table_hbm, src_hbm, dst_hbm, zeros_hbm, out_hbm,
          src_v, dst_v, rows, stage_v, agg_sh, gsem, ssem):
        cid = lax.axis_index("c")
        sid = lax.axis_index("s")
        wid = cid * NS + sid
        r0 = sid * ROWS_PER_TILE
        pltpu.sync_copy(zeros_hbm, stage_v)
        pltpu.sync_copy(stage_v, agg_sh.at[pl.ds(r0, ROWS_PER_TILE)])
        b0 = wid * NB_PER_TILE
        pltpu.sync_copy(src_hbm.at[pl.ds(b0, NB_PER_TILE)], src_v)
        pltpu.sync_copy(dst_hbm.at[pl.ds(b0, NB_PER_TILE)], dst_v)
        plsc.subcore_barrier()

        def gather(j, b):
            pltpu.async_copy(table_hbm.at[src_v.at[j]], rows[b], gsem[b])

        def wait_gather(j, b):
            pltpu.make_async_copy(
                table_hbm.at[src_v.at[j]], rows[b], gsem[b]).wait()

        def scatter(j, b):
            pltpu.async_copy(
                rows[b], agg_sh.at[dst_v.at[j]], ssem[b], add=True)

        def wait_scatter(j, b):
            pltpu.make_async_copy(
                rows[b], agg_sh.at[dst_v.at[j]], ssem[b]).wait()

        for b in range(NBUF):
            gather(b, b)
        @pl.loop(0, NB_PER_TILE - NBUF, step=NBUF)
        def _(i):
            for b in range(NBUF):
                wait_gather(i + b, b)
                scatter(i + b, b)
            for b in range(NBUF):
                wait_scatter(i + b, b)
                gather(i + NBUF + b, b)
        for b in range(NBUF):
            j = NB_PER_TILE - NBUF + b
            wait_gather(j, b)
            scatter(j, b)
        for b in range(NBUF):
            wait_scatter(NB_PER_TILE - NBUF + b, b)

        plsc.subcore_barrier()
        pltpu.sync_copy(agg_sh.at[pl.ds(r0, ROWS_PER_TILE)], stage_v)
        pltpu.sync_copy(stage_v, out_hbm.at[cid, pl.ds(r0, ROWS_PER_TILE)])

    return k


_make_sc_segsum = functools.lru_cache(maxsize=None)(_make_sc_segsum)


def kernel(x, edge_index, y, W1_rel, b1_rel, W1_root, W2_rel, b2_rel, W2_root):
    x_p = jnp.pad(x, ((0, NPAD - N), (0, 0)))
    pad_e = EPAD - E
    pad_idx = N + jnp.arange(pad_e, dtype=jnp.int32) % (NPAD - N)
    src_p = jnp.concatenate([edge_index[0], pad_idx]).reshape(-1, EB)
    dst_p = jnp.concatenate([edge_index[1], pad_idx]).reshape(-1, EB)
    y2d = jnp.pad(y, (0, NPAD - N)).reshape(NPAD, 1)
    b1_2d = b1_rel.reshape(1, H)
    w2rel_p = jnp.pad(W2_rel, ((0, CP - C), (0, 0)))
    w2root_p = jnp.pad(W2_root, ((0, CP - C), (0, 0)))
    b2_2d = jnp.pad(b2_rel, (0, CP - C)).reshape(1, CP)
    zeros_h = jnp.zeros((ROWS_PER_TILE, H), jnp.float32)
    zeros_c = jnp.zeros((ROWS_PER_TILE, CP), jnp.float32)

    xr, xroot = _tc_project(x_p, W1_rel, W1_root)
    part1 = _make_sc_segsum(H)(xr, src_p, dst_p, zeros_h)
    hr, hroot = _tc_layer2(part1[0], part1[1], xroot, b1_2d,
                           w2rel_p, w2root_p)
    part2 = _make_sc_segsum(CP)(hr, src_p, dst_p, zeros_c)
    loss2d = _tc_loss(part2[0], part2[1], hroot, b2_2d, y2d)
    return (loss2d[0, 0],)


# speedup vs baseline: 20.5305x; 1.0752x over previous
import functools

import jax
import jax.numpy as jnp
from jax import lax
from jax.experimental import pallas as pl
from jax.experimental.pallas import tpu as pltpu
from jax.experimental.pallas import tpu_sc as plsc

N = 10000
E = 320000
D = 128
H = 32
C = 10
CP = 16
NPAD = 10240
NC = 2
NS = 16
NW = NC * NS
EB = 128
NBUF = 8
NB_PER_TILE = 8 * -(-E // (NW * EB * 8))
EPAD = NW * EB * NB_PER_TILE
ROWS_PER_TILE = NPAD // NS


def _tc_project(x, wa, wb):
    def body(x_ref, wa_ref, wb_ref, oa_ref, ob_ref):
        xv = x_ref[...]
        dn = (((1,), (1,)), ((), ()))
        oa_ref[...] = lax.dot_general(xv, wa_ref[...], dn,
                                      preferred_element_type=jnp.float32)
        ob_ref[...] = lax.dot_general(xv, wb_ref[...], dn,
                                      preferred_element_type=jnp.float32)
    m = x.shape[0]
    return pl.pallas_call(
        body,
        out_shape=(jax.ShapeDtypeStruct((m, wa.shape[0]), jnp.float32),
                   jax.ShapeDtypeStruct((m, wb.shape[0]), jnp.float32)),
    )(x, wa, wb)


def _tc_layer2(p0, p1, xroot, b1, w2rel, w2root):
    def body(p0_ref, p1_ref, xroot_ref, b1_ref, wr_ref, wo_ref,
             hr_ref, hroot_ref):
        h = jnp.maximum(
            p0_ref[...] + p1_ref[...] + xroot_ref[...] + b1_ref[...], 0.0)
        dn = (((1,), (1,)), ((), ()))
        hr_ref[...] = lax.dot_general(h, wr_ref[...], dn,
                                      preferred_element_type=jnp.float32)
        hroot_ref[...] = lax.dot_general(h, wo_ref[...], dn,
                                         preferred_element_type=jnp.float32)
    m = p0.shape[0]
    return pl.pallas_call(
        body,
        out_shape=(jax.ShapeDtypeStruct((m, CP), jnp.float32),
                   jax.ShapeDtypeStruct((m, CP), jnp.float32)),
    )(p0, p1, xroot, b1, w2rel, w2root)


def _tc_loss(p0, p1, hroot, b2, y2d):
    def body(p0_ref, p1_ref, hroot_ref, b2_ref, y_ref, o_ref):
        logits = p0_ref[...] + p1_ref[...] + hroot_ref[...] + b2_ref[...]
        col = lax.broadcasted_iota(jnp.int32, logits.shape, 1)
        lm = jnp.where(col < C, logits, -1e30)
        mx = jnp.max(lm, axis=1, keepdims=True)
        ex = jnp.exp(lm - mx)
        lse = jnp.log(jnp.sum(ex, axis=1, keepdims=True)) + mx
        picked = jnp.sum(jnp.where(col == y_ref[...], lm, 0.0),
                         axis=1, keepdims=True)
        nll = lse - picked
        row = lax.broadcasted_iota(jnp.int32, nll.shape, 0)
        nll = jnp.where(row < N, nll, 0.0)
        o_ref[...] = (jnp.sum(nll) * (1.0 / N)).reshape(1, 1)
    return pl.pallas_call(
        body,
        out_shape=jax.ShapeDtypeStruct((1, 1), jnp.float32),
    )(p0, p1, hroot, b2, y2d)


def _make_sc_segsum(width):
    mesh = plsc.VectorSubcoreMesh(
        core_axis_name="c", subcore_axis_name="s",
        num_cores=NC, num_subcores=NS)

    @functools.partial(
        pl.kernel, mesh=mesh,
        out_type=jax.ShapeDtypeStruct((NC, NPAD, width), jnp.float32),
        scratch_types=[
            pltpu.VMEM((NB_PER_TILE, EB), jnp.int32),
            pltpu.VMEM((NB_PER_TILE, EB), jnp.int32),
            [pltpu.VMEM((EB, width), jnp.float32)] * NBUF,
            pltpu.VMEM((ROWS_PER_TILE, width), jnp.float32),
            pltpu.VMEM_SHARED((NPAD, width), jnp.float32),
            [pltpu.SemaphoreType.DMA] * NBUF,
            [pltpu.SemaphoreType.DMA] * NBUF,
        ],
        compiler_params=pltpu.CompilerParams(use_tc_tiling_on_sc=False),
    )
    def k(table_hbm, src_hbm, dst_hbm, zeros_hbm, out_hbm,
          src_v, dst_v, rows, stage_v, agg_sh, gsem, ssem):
        cid = lax.axis_index("c")
        sid = lax.axis_index("s")
        wid = cid * NS + sid
        r0 = sid * ROWS_PER_TILE
        b0 = wid * NB_PER_TILE
        pltpu.async_copy(src_hbm.at[pl.ds(b0, NB_PER_TILE)], src_v, gsem[0])
        pltpu.async_copy(dst_hbm.at[pl.ds(b0, NB_PER_TILE)], dst_v, gsem[1])
        pltpu.sync_copy(zeros_hbm, stage_v)
        pltpu.sync_copy(stage_v, agg_sh.at[pl.ds(r0, ROWS_PER_TILE)])
        pltpu.make_async_copy(
            src_hbm.at[pl.ds(b0, NB_PER_TILE)], src_v, gsem[0]).wait()
        pltpu.make_async_copy(
            dst_hbm.at[pl.ds(b0, NB_PER_TILE)], dst_v, gsem[1]).wait()
        plsc.subcore_barrier()

        def gather(j, b):
            pltpu.async_copy(table_hbm.at[src_v.at[j]], rows[b], gsem[b])

        def wait_gather(j, b):
            pltpu.make_async_copy(
                table_hbm.at[src_v.at[j]], rows[b], gsem[b]).wait()

        def scatter(j, b):
            pltpu.async_copy(
                rows[b], agg_sh.at[dst_v.at[j]], ssem[b], add=True)

        def wait_scatter(j, b):
            pltpu.make_async_copy(
                rows[b], agg_sh.at[dst_v.at[j]], ssem[b]).wait()

        for b in range(NBUF):
            gather(b, b)
        @pl.loop(0, NB_PER_TILE - NBUF, step=NBUF)
        def _(i):
            for b in range(NBUF):
                wait_gather(i + b, b)
                scatter(i + b, b)
            for b in range(NBUF):
                wait_scatter(i + b, b)
                gather(i + NBUF + b, b)
        for b in range(NBUF):
            j = NB_PER_TILE - NBUF + b
            wait_gather(j, b)
            scatter(j, b)
        for b in range(NBUF):
            wait_scatter(NB_PER_TILE - NBUF + b, b)

        plsc.subcore_barrier()
        pltpu.sync_copy(agg_sh.at[pl.ds(r0, ROWS_PER_TILE)], stage_v)
        pltpu.sync_copy(stage_v, out_hbm.at[cid, pl.ds(r0, ROWS_PER_TILE)])

    return k


_make_sc_segsum = functools.lru_cache(maxsize=None)(_make_sc_segsum)


def kernel(x, edge_index, y, W1_rel, b1_rel, W1_root, W2_rel, b2_rel, W2_root):
    x_p = jnp.pad(x, ((0, NPAD - N), (0, 0)))
    pad_e = EPAD - E
    pad_idx = N + jnp.arange(pad_e, dtype=jnp.int32) % (NPAD - N)
    src_p = jnp.concatenate([edge_index[0], pad_idx]).reshape(-1, EB)
    dst_p = jnp.concatenate([edge_index[1], pad_idx]).reshape(-1, EB)
    y2d = jnp.pad(y, (0, NPAD - N)).reshape(NPAD, 1)
    b1_2d = b1_rel.reshape(1, H)
    w2rel_p = jnp.pad(W2_rel, ((0, CP - C), (0, 0)))
    w2root_p = jnp.pad(W2_root, ((0, CP - C), (0, 0)))
    b2_2d = jnp.pad(b2_rel, (0, CP - C)).reshape(1, CP)
    zeros_h = jnp.zeros((ROWS_PER_TILE, H), jnp.float32)
    zeros_c = jnp.zeros((ROWS_PER_TILE, CP), jnp.float32)

    xr, xroot = _tc_project(x_p, W1_rel, W1_root)
    part1 = _make_sc_segsum(H)(xr, src_p, dst_p, zeros_h)
    hr, hroot = _tc_layer2(part1[0], part1[1], xroot, b1_2d,
                           w2rel_p, w2root_p)
    part2 = _make_sc_segsum(CP)(hr, src_p, dst_p, zeros_c)
    loss2d = _tc_loss(part2[0], part2[1], hroot, b2_2d, y2d)
    return (loss2d[0, 0],)
